# baseline (speedup 1.0000x reference)
import jax
import jax.numpy as jnp
from jax import lax
from jax.experimental import pallas as pl
from jax.experimental.pallas import tpu as pltpu
from jax.experimental.pallas import tpu_sc as plsc

F = 26
B = 4096
D = 32
V1 = 100001
NC = 2
NS = 16
NW = NC * NS
BPW = B // NW
W2SZ = F * D * V1


def _sc_gather(gidx_hbm, tab_hbm, w1t_hbm, x_out, y1_out,
               gidx_v, xstage, e1tmp, y1_v, semx, seme):
    wid = lax.axis_index("s") * NC + lax.axis_index("c")
    pltpu.sync_copy(gidx_hbm.at[wid], gidx_v)
    zeros16 = jnp.zeros((16,), jnp.float32)
    for m in range(BPW // 16):
        y1_v[pl.ds(m * 16, 16)] = zeros16

    def feature_body(f, carry):
        idx = gidx_v.at[f]
        for d in range(D):
            pltpu.async_copy(tab_hbm.at[f * D + d].at[idx], xstage.at[d], semx)
        pltpu.async_copy(w1t_hbm.at[f].at[idx], e1tmp, seme)
        pltpu.make_async_copy(w1t_hbm.at[0].at[pl.ds(0, BPW)],
                              e1tmp, seme).wait()
        for m in range(BPW // 16):
            plsc.addupdate(y1_v.at[pl.ds(m * 16, 16)], e1tmp[pl.ds(m * 16, 16)])
        pltpu.make_async_copy(x_out.at[wid, f], xstage, semx).wait()
        pltpu.sync_copy(xstage, x_out.at[wid, f])
        return carry

    lax.fori_loop(0, F, feature_body, 0)
    pltpu.sync_copy(y1_v, y1_out.at[wid])


def _ln(x, g, b, eps=1e-5):
    mu = jnp.mean(x, axis=-1, keepdims=True)
    var = jnp.mean((x - mu) ** 2, axis=-1, keepdims=True)
    return (x - mu) / jnp.sqrt(var + eps) * g + b


def _tc_body(x_ref, y1_ref, W0_ref, b0_ref, g0_ref, be0_ref,
             W1_ref, b1_ref, g1_ref, be1_ref, w2r_ref, b2_ref, o_ref):
    xcat = x_ref[0].reshape(F * D, BPW)
    h = lax.dot_general(xcat, W0_ref[...], (((0,), (0,)), ((), ())),
                        preferred_element_type=jnp.float32) + b0_ref[...]
    h = jnp.maximum(_ln(h, g0_ref[...], be0_ref[...]), 0.0)
    h = jnp.dot(h, W1_ref[...], preferred_element_type=jnp.float32) + b1_ref[...]
    h = jnp.maximum(_ln(h, g1_ref[...], be1_ref[...]), 0.0)
    y_dnn = jnp.sum(h * w2r_ref[...], axis=-1, keepdims=True) + b2_ref[...]
    s = x_ref[0, 0]
    for f in range(1, F):
        s = s + x_ref[0, f]
    y2row = 0.5 * (jnp.sum(s * s, axis=0, keepdims=True)
                   - jnp.sum(xcat * xcat, axis=0, keepdims=True))
    yrow = y1_ref[0] + y2row
    o_ref[...] = jax.nn.sigmoid(jnp.transpose(yrow) + y_dnn)


def kernel(indices, w1, w2, W0, b0, g0, be0, W1, b1, g1, be1, W2, b2):
    tab = w2.transpose(0, 2, 1).reshape(F * D, V1)
    w1t = w1.transpose(0, 2, 1).reshape(F, V1)
    gidx = indices.astype(jnp.int32).reshape(F, NW, BPW).swapaxes(0, 1)

    sc = pl.kernel(
        _sc_gather,
        out_type=[
            jax.ShapeDtypeStruct((NW, F, D, BPW), jnp.float32),
            jax.ShapeDtypeStruct((NW, BPW), jnp.float32),
        ],
        mesh=plsc.VectorSubcoreMesh(core_axis_name="c", subcore_axis_name="s"),
        scratch_types=[
            pltpu.VMEM((F, BPW), jnp.int32),
            pltpu.VMEM((D, BPW), jnp.float32),
            pltpu.VMEM((BPW,), jnp.float32),
            pltpu.VMEM((BPW,), jnp.float32),
            pltpu.SemaphoreType.DMA,
            pltpu.SemaphoreType.DMA,
        ],
        compiler_params=pltpu.CompilerParams(needs_layout_passes=False,
                                             use_tc_tiling_on_sc=False),
    )
    x4, y1o = sc(gidx, tab, w1t)

    grid = NW
    out2 = pl.pallas_call(
        _tc_body,
        grid=(grid,),
        in_specs=[
            pl.BlockSpec((1, F, D, BPW), lambda i: (i, 0, 0, 0)),
            pl.BlockSpec((1, 1, BPW), lambda i: (i, 0, 0)),
            pl.BlockSpec(W0.shape, lambda i: (0, 0)),
            pl.BlockSpec((1, b0.shape[0]), lambda i: (0, 0)),
            pl.BlockSpec((1, g0.shape[0]), lambda i: (0, 0)),
            pl.BlockSpec((1, be0.shape[0]), lambda i: (0, 0)),
            pl.BlockSpec(W1.shape, lambda i: (0, 0)),
            pl.BlockSpec((1, b1.shape[0]), lambda i: (0, 0)),
            pl.BlockSpec((1, g1.shape[0]), lambda i: (0, 0)),
            pl.BlockSpec((1, be1.shape[0]), lambda i: (0, 0)),
            pl.BlockSpec((1, W2.shape[0]), lambda i: (0, 0)),
            pl.BlockSpec((1, 1), lambda i: (0, 0)),
        ],
        out_specs=pl.BlockSpec((BPW, 1), lambda i: (i, 0)),
        out_shape=jax.ShapeDtypeStruct((B, 1), jnp.float32),
    )(x4, y1o.reshape(NW, 1, BPW), W0,
      b0.reshape(1, -1), g0.reshape(1, -1), be0.reshape(1, -1),
      W1, b1.reshape(1, -1), g1.reshape(1, -1), be1.reshape(1, -1),
      W2.reshape(1, -1), b2.reshape(1, 1))
    return out2[:, 0]
